# baseline (speedup 1.0000x reference)
import functools

import jax
import jax.numpy as jnp
from jax import lax
from jax.experimental import pallas as pl
from jax.experimental.pallas import tpu as pltpu
from jax.experimental.pallas import tpu_sc as plsc

_N_VOCAB = 1000000
_N_EMBED = 64
_BATCH = 16384
_WIDE = 2 * _N_EMBED

_NUM_CORES = 2
_NUM_SUBCORES = 16
_NUM_WORKERS = _NUM_CORES * _NUM_SUBCORES
_B_PER_W = _BATCH // _NUM_WORKERS
_CHUNK = 128
_N_CHUNKS = _B_PER_W // _CHUNK
_OUT_ROWS_PER_W = _B_PER_W // 2

_mesh = plsc.VectorSubcoreMesh(core_axis_name="c", subcore_axis_name="s")


@functools.partial(
    pl.kernel,
    mesh=_mesh,
    out_type=jax.ShapeDtypeStruct((_BATCH // 2, _WIDE), jnp.float32),
    scratch_types=[
        pltpu.VMEM((_N_CHUNKS, _CHUNK), jnp.int32),
        pltpu.VMEM((_N_CHUNKS, _CHUNK), jnp.int32),
        pltpu.VMEM((_B_PER_W, _WIDE), jnp.float32),
        pltpu.SemaphoreType.DMA,
    ],
)
def _sc_gather(idx_hbm, table_hbm, out_hbm, idx_v, q_v, rows_v, sem):
    wid = lax.axis_index("s") * _NUM_CORES + lax.axis_index("c")
    pltpu.sync_copy(idx_hbm.at[pl.ds(wid * _N_CHUNKS, _N_CHUNKS)], idx_v)

    for r in range(_N_CHUNKS):
        for c in range(_CHUNK // 16):
            q_v[r, pl.ds(c * 16, 16)] = lax.shift_right_logical(
                idx_v[r, pl.ds(c * 16, 16)], 1
            )

    copies = [
        pltpu.async_copy(
            table_hbm.at[q_v.at[j]],
            rows_v.at[pl.ds(j * _CHUNK, _CHUNK)],
            sem,
        )
        for j in range(_N_CHUNKS)
    ]
    for cp in copies:
        cp.wait()

    def select(r, carry):
        for g in range(_CHUNK // 16):
            s_vec = idx_v[r, pl.ds(g * 16, 16)] & 1
            for j in range(16):
                jj = g * 16 + j
                src_row = r * _CHUNK + jj
                dst_row = r * (_CHUNK // 2) + jj // 2
                dst_col = (jj % 2) * _N_EMBED
                off = s_vec[j] * _N_EMBED
                for c in range(_N_EMBED // 16):
                    rows_v[dst_row, pl.ds(dst_col + c * 16, 16)] = rows_v[
                        src_row, pl.ds(off + c * 16, 16)
                    ]
        return carry

    lax.fori_loop(0, _N_CHUNKS, select, 0)

    pltpu.sync_copy(
        rows_v.at[pl.ds(0, _OUT_ROWS_PER_W)],
        out_hbm.at[pl.ds(wid * _OUT_ROWS_PER_W, _OUT_ROWS_PER_W)],
    )


def kernel(input_words, in_embed):
    idx = input_words.astype(jnp.int32).reshape(
        _NUM_WORKERS * _N_CHUNKS, _CHUNK
    )
    table_wide = in_embed.reshape(_N_VOCAB // 2, _WIDE)
    out_wide = _sc_gather(idx, table_wide)
    return out_wide.reshape(_BATCH, _N_EMBED)


# speedup vs baseline: 1.7502x; 1.7502x over previous
import functools

import jax
import jax.numpy as jnp
from jax import lax
from jax.experimental import pallas as pl
from jax.experimental.pallas import tpu as pltpu
from jax.experimental.pallas import tpu_sc as plsc

_N_VOCAB = 1000000
_N_EMBED = 64
_BATCH = 16384

_NUM_CORES = 2
_NUM_SUBCORES = 16
_NUM_WORKERS = _NUM_CORES * _NUM_SUBCORES
_B_PER_W = _BATCH // _NUM_WORKERS

_mesh = plsc.VectorSubcoreMesh(core_axis_name="c", subcore_axis_name="s")


@functools.partial(
    pl.kernel,
    mesh=_mesh,
    out_type=jax.ShapeDtypeStruct((_BATCH, _N_EMBED), jnp.float32),
    scratch_types=[
        pltpu.VMEM((_B_PER_W,), jnp.int32),
        pltpu.VMEM((_B_PER_W, _N_EMBED), jnp.float32),
        pltpu.SemaphoreType.DMA,
    ],
)
def _sc_gather(idx_hbm, table_hbm, out_hbm, idx_v, rows_v, sem):
    wid = lax.axis_index("s") * _NUM_CORES + lax.axis_index("c")
    base = wid * _B_PER_W
    pltpu.sync_copy(idx_hbm.at[pl.ds(base, _B_PER_W)], idx_v)

    def fire(g, carry):
        v = idx_v[pl.ds(g * 16, 16)]
        for j in range(16):
            p = v[j]
            pltpu.async_copy(
                table_hbm.at[pl.ds(p, 1)],
                rows_v.at[pl.ds(g * 16 + j, 1)],
                sem,
            )
        return carry

    lax.fori_loop(0, _B_PER_W // 16, fire, 0)

    def drain(i, carry):
        pltpu.make_async_copy(
            table_hbm.at[pl.ds(0, 1)],
            rows_v.at[pl.ds(0, 1)],
            sem,
        ).wait()
        return carry

    lax.fori_loop(0, _B_PER_W, drain, 0)

    pltpu.sync_copy(rows_v, out_hbm.at[pl.ds(base, _B_PER_W)])


def kernel(input_words, in_embed):
    idx = input_words.astype(jnp.int32)
    return _sc_gather(idx, in_embed)


# speedup vs baseline: 1.7552x; 1.0029x over previous
import functools

import jax
import jax.numpy as jnp
from jax import lax
from jax.experimental import pallas as pl
from jax.experimental.pallas import tpu as pltpu
from jax.experimental.pallas import tpu_sc as plsc

_N_VOCAB = 1000000
_N_EMBED = 64
_BATCH = 16384

_NUM_CORES = 2
_NUM_SUBCORES = 16
_NUM_WORKERS = _NUM_CORES * _NUM_SUBCORES
_B_PER_W = _BATCH // _NUM_WORKERS

_mesh = plsc.VectorSubcoreMesh(core_axis_name="c", subcore_axis_name="s")


@functools.partial(
    pl.kernel,
    mesh=_mesh,
    out_type=jax.ShapeDtypeStruct((_BATCH, _N_EMBED), jnp.float32),
    scratch_types=[
        pltpu.VMEM((_B_PER_W,), jnp.int32),
        pltpu.VMEM((_B_PER_W, _N_EMBED), jnp.float32),
        pltpu.SemaphoreType.DMA,
    ],
    compiler_params=pltpu.CompilerParams(
        skip_device_barrier=True,
        disable_semaphore_checks=True,
    ),
)
def _sc_gather(idx_hbm, table_hbm, out_hbm, idx_v, rows_v, sem):
    wid = lax.axis_index("s") * _NUM_CORES + lax.axis_index("c")
    base = wid * _B_PER_W
    pltpu.sync_copy(idx_hbm.at[pl.ds(base, _B_PER_W)], idx_v)

    def fire(g, carry):
        v = idx_v[pl.ds(g * 16, 16)]
        for j in range(16):
            p = v[j]
            pltpu.async_copy(
                table_hbm.at[pl.ds(p, 1)],
                rows_v.at[pl.ds(g * 16 + j, 1)],
                sem,
            )
        return carry

    lax.fori_loop(0, _B_PER_W // 16, fire, 0)

    def drain(i, carry):
        pltpu.make_async_copy(
            table_hbm.at[pl.ds(0, 1)],
            rows_v.at[pl.ds(0, 1)],
            sem,
        ).wait()
        return carry

    lax.fori_loop(0, _B_PER_W, drain, 0)

    pltpu.sync_copy(rows_v, out_hbm.at[pl.ds(base, _B_PER_W)])


def kernel(input_words, in_embed):
    idx = input_words.astype(jnp.int32)
    return _sc_gather(idx, in_embed)
